# baseline (speedup 1.0000x reference)
import jax
import jax.numpy as jnp
from jax import lax
from jax.experimental import pallas as pl
from jax.experimental.pallas import tpu as pltpu
from jax.experimental.pallas import tpu_sc as plsc

_B = 16384
_D = 64
_NC, _NS, _L = 2, 16, 16
_NW = _NC * _NS
_BPW = _B // _NW
_CH = 128
_NCH = _BPW // _CH


def _sc_body(uidx_hbm, iidx_hbm, eu_hbm, ei_hbm, bias_hbm, out_hbm,
             uidx_v, iidx_v, urows_v, irows_v, bias_v, out_v, sem):
    wid = lax.axis_index("s") * _NC + lax.axis_index("c")
    base = wid * _BPW
    pltpu.sync_copy(uidx_hbm.at[pl.ds(base, _BPW)], uidx_v)
    pltpu.sync_copy(iidx_hbm.at[pl.ds(base, _BPW)], iidx_v)

    copies = []
    for j in range(_NCH):
        s = pl.ds(j * _CH, _CH)
        copies.append(pltpu.async_copy(eu_hbm.at[uidx_v.at[s]], urows_v.at[s], sem))
        copies.append(pltpu.async_copy(ei_hbm.at[iidx_v.at[s]], irows_v.at[s], sem))
        copies.append(pltpu.async_copy(bias_hbm.at[iidx_v.at[s]], bias_v.at[s], sem))
    for c in copies:
        c.wait()

    iota16 = lax.iota(jnp.int32, _L)

    def group_body(g, carry):
        rows = g * _L + iota16
        bias16 = bias_v[pl.ds(g * _L, _L)]

        def d_body(d, acc):
            cols = jnp.full((_L,), d, jnp.int32)
            u = plsc.load_gather(urows_v, [rows, cols])
            it = plsc.load_gather(irows_v, [rows, cols])
            return acc + u * it

        acc = lax.fori_loop(0, _D, d_body, bias16)
        out_v[pl.ds(g * _L, _L)] = acc
        return carry

    lax.fori_loop(0, _BPW // _L, group_body, 0)
    pltpu.sync_copy(out_v, out_hbm.at[pl.ds(base, _BPW)])


def kernel(user_indices, item_indices, embedding_user, embedding_item, bias_item):
    ui = user_indices.astype(jnp.int32)
    ii = item_indices.astype(jnp.int32)
    mesh = plsc.VectorSubcoreMesh(core_axis_name="c", subcore_axis_name="s")
    f = pl.kernel(
        _sc_body,
        out_type=jax.ShapeDtypeStruct((_B,), jnp.float32),
        mesh=mesh,
        compiler_params=pltpu.CompilerParams(
            needs_layout_passes=False, use_tc_tiling_on_sc=False
        ),
        scratch_types=[
            pltpu.VMEM((_BPW,), jnp.int32),
            pltpu.VMEM((_BPW,), jnp.int32),
            pltpu.VMEM((_BPW, _D), jnp.float32),
            pltpu.VMEM((_BPW, _D), jnp.float32),
            pltpu.VMEM((_BPW,), jnp.float32),
            pltpu.VMEM((_BPW,), jnp.float32),
            pltpu.SemaphoreType.DMA,
        ],
    )
    return f(ui, ii, embedding_user, embedding_item, bias_item.reshape(-1))
